# baseline (speedup 1.0000x reference)
import functools

import jax
import jax.numpy as jnp
from jax import lax
from jax.experimental import pallas as pl
from jax.experimental.pallas import tpu as pltpu
from jax.experimental.pallas import tpu_sc as plsc

B = 16384
D = 64
NC = 2
NS = 16
NW = NC * NS
BPW = B // NW

_sc_mesh = plsc.VectorSubcoreMesh(core_axis_name="c", subcore_axis_name="s")


@functools.partial(
    pl.kernel,
    out_type=(
        jax.ShapeDtypeStruct((B, D), jnp.float32),
        jax.ShapeDtypeStruct((B, D), jnp.float32),
    ),
    mesh=_sc_mesh,
    compiler_params=pltpu.CompilerParams(use_tc_tiling_on_sc=False),
    scratch_types=[
        pltpu.VMEM((BPW,), jnp.int32),
        pltpu.VMEM((BPW,), jnp.int32),
        pltpu.VMEM((BPW, D), jnp.float32),
        pltpu.VMEM((BPW, D), jnp.float32),
        pltpu.SemaphoreType.DMA,
        pltpu.SemaphoreType.DMA,
    ],
)
def _sc_gather2(uid_hbm, iid_hbm, ut_hbm, it_hbm, ue_hbm, ie_hbm,
                uidx_v, iidx_v, urows_v, irows_v, sem_u, sem_i):
    wid = lax.axis_index("s") * NC + lax.axis_index("c")
    base = wid * BPW
    pltpu.sync_copy(uid_hbm.at[pl.ds(base, BPW)], uidx_v)
    pltpu.sync_copy(iid_hbm.at[pl.ds(base, BPW)], iidx_v)
    cu = pltpu.async_copy(ut_hbm.at[uidx_v], urows_v, sem_u)
    ci = pltpu.async_copy(it_hbm.at[iidx_v], irows_v, sem_i)
    cu.wait()
    ci.wait()
    pltpu.sync_copy(urows_v, ue_hbm.at[pl.ds(base, BPW)])
    pltpu.sync_copy(irows_v, ie_hbm.at[pl.ds(base, BPW)])


BLK = 2048


def _mlp_body(ue_ref, ie_ref, w1u_ref, w1i_ref, b1_ref, w2_ref, b2_ref,
              w3_ref, b3_ref, out_ref):
    x1 = (jnp.dot(ue_ref[...], w1u_ref[...], preferred_element_type=jnp.float32)
          + jnp.dot(ie_ref[...], w1i_ref[...], preferred_element_type=jnp.float32)
          + b1_ref[...])
    h1 = jnp.maximum(x1, 0.0)
    h2 = jnp.maximum(
        jnp.dot(h1, w2_ref[...], preferred_element_type=jnp.float32) + b2_ref[...],
        0.0)
    z = jnp.sum(h2 * w3_ref[...], axis=1, keepdims=True) + b3_ref[0, 0]
    out_ref[...] = 1.0 / (1.0 + jnp.exp(-z))


def _mlp(ue, ie, w1u, w1i, b1r, w2t, b2r, w3r, b3r):
    grid = (B // BLK,)
    full = lambda shape: pl.BlockSpec(shape, lambda i: (0, 0))
    return pl.pallas_call(
        _mlp_body,
        grid=grid,
        in_specs=[
            pl.BlockSpec((BLK, D), lambda i: (i, 0)),
            pl.BlockSpec((BLK, D), lambda i: (i, 0)),
            full((D, 128)),
            full((D, 128)),
            full((1, 128)),
            full((128, D)),
            full((1, D)),
            full((1, D)),
            full((1, 1)),
        ],
        out_specs=pl.BlockSpec((BLK, 1), lambda i: (i, 0)),
        out_shape=jax.ShapeDtypeStruct((B, 1), jnp.float32),
    )(ue, ie, w1u, w1i, b1r, w2t, b2r, w3r, b3r)


def kernel(user_ids, item_ids, user_table, item_table, W1, b1, W2, b2, W3, b3):
    uid = user_ids.astype(jnp.int32)
    iid = item_ids.astype(jnp.int32)
    ue, ie = _sc_gather2(uid, iid, user_table, item_table)
    w1u = W1[:, :D].T
    w1i = W1[:, D:].T
    w2t = W2.T
    out = _mlp(ue, ie, w1u, w1i, b1.reshape(1, 128), w2t, b2.reshape(1, D),
               W3, b3.reshape(1, 1))
    return out[:, 0]
